# baseline (speedup 1.0000x reference)
import functools

import jax
import jax.numpy as jnp
import numpy as np
from jax import lax
from jax.experimental import pallas as pl
from jax.experimental.pallas import tpu as pltpu
from jax.experimental.pallas import tpu_sc as plsc

_CARDINALITIES = [100000] * 26
_F = len(_CARDINALITIES)
_D = 32
_B = 16384
_TOTAL = _B * _F
_NW = 32
_PER_W = _TOTAL // _NW
_PAT = 208
_C = 1664
_NCH = _PER_W // _C
_GRP = _C // _PAT
_GB = 128
_NG = _C // _GB


def _build_patterns():
    offsets = np.concatenate(
        [np.zeros((1,), np.int32), np.cumsum(np.asarray(_CARDINALITIES[:-1], np.int32))]
    ).astype(np.int32)
    i = np.arange(_PAT, dtype=np.int32)
    f_pat = i % _F
    off_pat = offsets[f_pat]
    return off_pat, f_pat.astype(np.int32)


_OFF_PAT, _F_PAT = _build_patterns()


def _make_kernel():
    mesh = plsc.VectorSubcoreMesh(core_axis_name="c", subcore_axis_name="s")

    @functools.partial(
        pl.kernel,
        mesh=mesh,
        out_type=jax.ShapeDtypeStruct((_TOTAL, _D), jnp.float32),
        compiler_params=pltpu.CompilerParams(use_tc_tiling_on_sc=False),
        scratch_types=[
            pltpu.VMEM((_C,), jnp.int32),
            pltpu.VMEM((_C,), jnp.int32),
            pltpu.VMEM((_C,), jnp.int32),
            pltpu.VMEM((_C,), jnp.int32),
            pltpu.VMEM((_C, _D), jnp.float32),
            pltpu.VMEM((_C, _D), jnp.float32),
            pltpu.VMEM((_PAT,), jnp.int32),
            pltpu.VMEM((_PAT,), jnp.int32),
            pltpu.SemaphoreType.DMA,
            pltpu.SemaphoreType.DMA,
        ],
    )
    def k(x_hbm, m_hbm, table_hbm, aug_hbm, offp_hbm, fp_hbm, out_hbm,
          x_v, m_v, idx_v, aidx_v, rows_v, add_v, offp_v, fp_v, sem_t, sem_a):
        nc = 2
        wid = lax.axis_index("s") * nc + lax.axis_index("c")
        pltpu.sync_copy(offp_hbm, offp_v)
        pltpu.sync_copy(fp_hbm, fp_v)

        def chunk_body(ch, carry):
            base = wid * _PER_W + ch * _C
            pltpu.sync_copy(x_hbm.at[pl.ds(base, _C)], x_v)
            pltpu.sync_copy(m_hbm.at[pl.ds(base, _C)], m_v)

            def idx_body(g, c2):
                s0 = g * _PAT
                for kk in range(_PAT // 16):
                    s = s0 + kk * 16
                    off16 = offp_v[pl.ds(kk * 16, 16)]
                    f16 = fp_v[pl.ds(kk * 16, 16)]
                    xv = x_v[pl.ds(s, 16)]
                    mv = m_v[pl.ds(s, 16)]
                    idx_v[pl.ds(s, 16)] = xv + off16
                    aidx_v[pl.ds(s, 16)] = f16 + mv * (_F - f16)
                return c2

            lax.fori_loop(0, _GRP, idx_body, 0)

            copies = []
            for i in range(_NG):
                sl = pl.ds(i * _GB, _GB)
                copies.append(
                    pltpu.async_copy(table_hbm.at[idx_v.at[sl]], rows_v.at[sl], sem_t)
                )
            for i in range(_NG):
                sl = pl.ds(i * _GB, _GB)
                copies.append(
                    pltpu.async_copy(aug_hbm.at[aidx_v.at[sl]], add_v.at[sl], sem_a)
                )
            for cp in copies:
                cp.wait()

            def row_body(b16, c2):
                keep16 = (
                    jnp.float32(1)
                    - m_v[pl.ds(b16 * 16, 16)].astype(jnp.float32)
                )
                for i in range(16):
                    r = b16 * 16 + i
                    keep = keep16[i]
                    for h in range(2):
                        sl = pl.ds(h * 16, 16)
                        rows_v[r, sl] = rows_v[r, sl] * keep + add_v[r, sl]
                return c2

            lax.fori_loop(0, _C // 16, row_body, 0)
            pltpu.sync_copy(rows_v, out_hbm.at[pl.ds(base, _C)])
            return carry

        lax.fori_loop(0, _NCH, chunk_body, 0)

    return k


_KERNEL = _make_kernel()


@jax.jit
def kernel(x, mask, table, bias, mask_weight):
    aug = jnp.concatenate([bias, mask_weight], axis=0)
    x_flat = x.reshape(-1)
    m_flat = mask.reshape(-1).astype(jnp.int32)
    off_pat = jnp.asarray(_OFF_PAT)
    f_pat = jnp.asarray(_F_PAT)
    out = _KERNEL(x_flat, m_flat, table, aug, off_pat, f_pat)
    return out.reshape(_B, _F, _D)


# speedup vs baseline: 2.0900x; 2.0900x over previous
import functools

import jax
import jax.numpy as jnp
import numpy as np
from jax import lax
from jax.experimental import pallas as pl
from jax.experimental.pallas import tpu as pltpu
from jax.experimental.pallas import tpu_sc as plsc

_CARDINALITIES = [100000] * 26
_F = len(_CARDINALITIES)
_D = 32
_B = 16384
_TOTAL = _B * _F
_NW = 32
_PER_W = _TOTAL // _NW
_PAT = 208
_C = 1664
_NCH = _PER_W // _C
_GRP = _C // _PAT
_GB = 128
_NG = _C // _GB


def _build_patterns():
    offsets = np.concatenate(
        [np.zeros((1,), np.int32), np.cumsum(np.asarray(_CARDINALITIES[:-1], np.int32))]
    ).astype(np.int32)
    i = np.arange(_PAT, dtype=np.int32)
    f_pat = i % _F
    off_pat = offsets[f_pat]
    return off_pat, f_pat.astype(np.int32)


_OFF_PAT, _F_PAT = _build_patterns()


def _make_kernel():
    mesh = plsc.VectorSubcoreMesh(core_axis_name="c", subcore_axis_name="s")

    @functools.partial(
        pl.kernel,
        mesh=mesh,
        out_type=jax.ShapeDtypeStruct((_TOTAL, _D), jnp.float32),
        compiler_params=pltpu.CompilerParams(use_tc_tiling_on_sc=False),
        scratch_types=[
            pltpu.VMEM((_C,), jnp.int32),
            pltpu.VMEM((_C,), jnp.int32),
            pltpu.VMEM((_C,), jnp.int32),
            pltpu.VMEM((_C, _D), jnp.float32),
            pltpu.VMEM((_F + 1, _D), jnp.float32),
            pltpu.VMEM((_PAT,), jnp.int32),
            pltpu.VMEM((_PAT,), jnp.int32),
            pltpu.SemaphoreType.DMA,
        ],
    )
    def k(x_hbm, m_hbm, table_hbm, aug_hbm, offp_hbm, fp_hbm, out_hbm,
          x_v, m_v, idx_v, rows_v, aug_v, offp_v, fp_v, sem_t):
        nc = 2
        wid = lax.axis_index("s") * nc + lax.axis_index("c")
        pltpu.sync_copy(offp_hbm, offp_v)
        pltpu.sync_copy(fp_hbm, fp_v)
        pltpu.sync_copy(aug_hbm, aug_v)

        def chunk_body(ch, carry):
            base = wid * _PER_W + ch * _C
            with jax.named_scope("in_dma"):
                pltpu.sync_copy(x_hbm.at[pl.ds(base, _C)], x_v)
                pltpu.sync_copy(m_hbm.at[pl.ds(base, _C)], m_v)

            with jax.named_scope("idx"):
                def idx_body(g, c2):
                    s0 = g * _PAT
                    for kk in range(_PAT // 16):
                        s = s0 + kk * 16
                        off16 = offp_v[pl.ds(kk * 16, 16)]
                        xv = x_v[pl.ds(s, 16)]
                        idx_v[pl.ds(s, 16)] = xv + off16
                    return c2

                lax.fori_loop(0, _GRP, idx_body, 0)

            with jax.named_scope("gather"):
                copies = []
                for i in range(_NG):
                    sl = pl.ds(i * _GB, _GB)
                    copies.append(
                        pltpu.async_copy(
                            table_hbm.at[idx_v.at[sl]], rows_v.at[sl], sem_t
                        )
                    )
                for cp in copies:
                    cp.wait()

            with jax.named_scope("compute"):
                def cmp_body(g, c2):
                    s0 = g * _PAT
                    for kk in range(_PAT // 16):
                        s = s0 + kk * 16
                        m16 = m_v[pl.ds(s, 16)]
                        f16 = fp_v[pl.ds(kk * 16, 16)]
                        keep16 = jnp.float32(1) - m16.astype(jnp.float32)
                        aidx16 = f16 + m16 * (_F - f16)
                        for i in range(16):
                            r = s + i
                            keep = keep16[i]
                            a = aidx16[i]
                            for h in range(2):
                                sl = pl.ds(h * 16, 16)
                                rows_v[r, sl] = (
                                    rows_v[r, sl] * keep + aug_v[a, sl]
                                )
                    return c2

                lax.fori_loop(0, _GRP, cmp_body, 0)

            with jax.named_scope("out_dma"):
                pltpu.sync_copy(rows_v, out_hbm.at[pl.ds(base, _C)])
            return carry

        lax.fori_loop(0, _NCH, chunk_body, 0)

    return k


_KERNEL = _make_kernel()


@jax.jit
def kernel(x, mask, table, bias, mask_weight):
    aug = jnp.concatenate([bias, mask_weight], axis=0)
    x_flat = x.reshape(-1)
    m_flat = mask.reshape(-1).astype(jnp.int32)
    off_pat = jnp.asarray(_OFF_PAT)
    f_pat = jnp.asarray(_F_PAT)
    out = _KERNEL(x_flat, m_flat, table, aug, off_pat, f_pat)
    return out.reshape(_B, _F, _D)
